# baseline (speedup 1.0000x reference)
import functools

import jax
import jax.numpy as jnp
from jax import lax
from jax.experimental import pallas as pl
from jax.experimental.pallas import tpu as pltpu
from jax.experimental.pallas import tpu_sc as plsc

DIM = 128
BATCH = 4096
HIST = 50
HPAD = 56
NCORES = 2
NSUB = 16
NW = NCORES * NSUB
BPW = BATCH // NW
LANES = 16

_SELU_ALPHA = 1.6732632423543772
_SELU_SCALE = 1.0507009873554805


def _sc_pool(idx_flat, table):
    mesh = plsc.VectorSubcoreMesh(
        core_axis_name="c", subcore_axis_name="s",
        num_cores=NCORES, num_subcores=NSUB,
    )

    @functools.partial(
        pl.kernel,
        out_type=jax.ShapeDtypeStruct((BATCH, DIM), jnp.float32),
        mesh=mesh,
        scratch_types=[
            pltpu.VMEM((BPW * HPAD,), jnp.int32),
            pltpu.VMEM((HPAD, DIM), jnp.float32),
            pltpu.VMEM((BPW, DIM), jnp.float32),
            pltpu.SemaphoreType.DMA,
        ],
    )
    def pool(idx_hbm, table_hbm, out_hbm, idx_v, buf, outbuf, sem):
        wid = lax.axis_index("c") * NSUB + lax.axis_index("s")
        base = wid * BPW
        pltpu.sync_copy(idx_hbm.at[pl.ds(base * HPAD, BPW * HPAD)], idx_v)

        def row_body(i, carry):
            pltpu.async_copy(
                table_hbm.at[idx_v.at[pl.ds(i * HPAD, HPAD)]], buf, sem
            ).wait()
            for d in range(DIM // LANES):
                sl = pl.ds(d * LANES, LANES)

                def inner(h, a):
                    return a + buf[h, sl]

                acc = lax.fori_loop(1, HIST, inner, buf[0, sl])
                outbuf[i, sl] = acc
            return carry

        lax.fori_loop(0, BPW, row_body, 0)
        pltpu.sync_copy(outbuf, out_hbm.at[pl.ds(base, BPW)])

    return pool(idx_flat, table)


def _selu_linear(x, wT, b2):
    def body(x_ref, w_ref, b_ref, o_ref):
        v = x_ref[...]
        v = _SELU_SCALE * jnp.where(v > 0, v, _SELU_ALPHA * (jnp.exp(v) - 1.0))
        o_ref[...] = (
            jnp.dot(v, w_ref[...], preferred_element_type=jnp.float32) + b_ref[...]
        )

    blk = 512
    return pl.pallas_call(
        body,
        out_shape=jax.ShapeDtypeStruct((BATCH, DIM), jnp.float32),
        grid=(BATCH // blk,),
        in_specs=[
            pl.BlockSpec((blk, DIM), lambda i: (i, 0)),
            pl.BlockSpec((DIM, DIM), lambda i: (0, 0)),
            pl.BlockSpec((1, DIM), lambda i: (0, 0)),
        ],
        out_specs=pl.BlockSpec((blk, DIM), lambda i: (i, 0)),
    )(x, wT, b2)


def kernel(input_text, table, W, b):
    idx = input_text.reshape(BATCH, -1).astype(jnp.int32)
    idx = jnp.pad(idx, ((0, 0), (0, HPAD - HIST)))
    pooled = _sc_pool(idx.reshape(-1), table)
    return _selu_linear(pooled, W.T, b.reshape(1, DIM))
